# baseline (speedup 1.0000x reference)
import functools

import jax
import jax.numpy as jnp
import numpy as np
from jax import lax
from jax.experimental import pallas as pl
from jax.experimental.pallas import tpu as pltpu
from jax.experimental.pallas import tpu_sc as plsc

B = 8
N = 10000
D = 128
U = 128
E = 320000
M = B * N

NUM_TECS = 16
CHUNK = 128
NCHUNK = 158
NCF = NCHUNK + 2
EPAD = NUM_TECS * NCF * CHUNK
RPT = 624
TAIL = N - RPT * NUM_TECS
VSCALE = float(2 ** 24)

_BM = 2000

_PERM = np.empty(U, np.int32)
for _j in range(U // 32):
    for _t in range(16):
        _PERM[32 * _j + 2 * _t] = 32 * _j + _t
        _PERM[32 * _j + 2 * _t + 1] = 32 * _j + 16 + _t


def _mm2_body(x_ref, k1_ref, k2_ref, z1_ref, z2_ref):
    xb = x_ref[...]
    z1_ref[...] = jnp.dot(xb, k1_ref[...], preferred_element_type=jnp.float32)
    z2_ref[...] = jnp.dot(xb, k2_ref[...], preferred_element_type=jnp.float32)


def _mmadd_body(x_ref, s_ref, k0_ref, b_ref, o_ref):
    o_ref[...] = (jnp.dot(x_ref[...], k0_ref[...],
                          preferred_element_type=jnp.float32)
                  + s_ref[...] + b_ref[...][0:1, :])


def _sc_body(z1_hbm, z2_hbm, p0_hbm, p1_hbm, out_hbm,
             acc, ring0, ring1, col0, col1,
             row0, row1, gbuf0, gbuf1, sem0, sem1):
    cid = lax.axis_index("c")
    sid = lax.axis_index("s")
    base = sid * RPT
    rings = (ring0, ring1)
    cols = (col0, col1)
    rows = (row0, row1)
    gbufs = (gbuf0, gbuf1)
    sems = (sem0, sem1)

    def batch_body(bi, _):
        b = cid * (B // 2) + bi
        bN = b * N

        def zloop(i, _):
            for j in range(U // 16):
                gbuf0[i, pl.ds(j * 16, 16)] = jnp.zeros((16,), jnp.float32)
            return 0
        lax.fori_loop(0, CHUNK, zloop, 0)
        for k in range(RPT // CHUNK):
            pltpu.sync_copy(gbuf0, acc.at[pl.ds(base + k * CHUNK, CHUNK)])
        rem = RPT % CHUNK
        if rem:
            pltpu.sync_copy(gbuf0.at[pl.ds(0, rem)],
                            acc.at[pl.ds(base + RPT - rem, rem)])

        @pl.when(sid == NUM_TECS - 1)
        def _zero_tail():
            pltpu.sync_copy(gbuf0.at[pl.ds(0, TAIL)],
                            acc.at[pl.ds(RPT * NUM_TECS, TAIL)])
        plsc.subcore_barrier()

        for z_hbm, p_hbm in ((z1_hbm, p0_hbm), (z2_hbm, p1_hbm)):

            def stage(k, r, z_hbm=z_hbm, p_hbm=p_hbm, bN=bN):
                rg = rings[r]
                cb = cols[r]
                pltpu.sync_copy(p_hbm.at[sid, k], rg)
                for j in range(CHUNK // 16):
                    cb[pl.ds(j * 16, 16)] = rg[0, pl.ds(j * 16, 16)] + bN
                pltpu.async_copy(z_hbm.at[cb], gbufs[r], sems[r])

            def process(k, r, z_hbm=z_hbm):
                pltpu.make_async_copy(
                    z_hbm.at[cols[r]], gbufs[r], sems[r]).wait()
                rg = rings[r]
                gb = gbufs[r]

                def srow(t, _):
                    valv = (rg[2, pl.ds(t * 16, 16)].astype(jnp.float32)
                            * (1.0 / VSCALE))
                    for i in range(16):
                        row = t * 16 + i
                        v = valv[i]
                        for j in range(U // 16):
                            gb[row, pl.ds(j * 16, 16)] = (
                                gb[row, pl.ds(j * 16, 16)] * v)
                    return 0
                lax.fori_loop(0, CHUNK // 16, srow, 0)
                rb = rows[r]
                for j in range(CHUNK // 16):
                    rb[pl.ds(j * 16, 16)] = rg[1, pl.ds(j * 16, 16)]
                pltpu.sync_copy(gb, acc.at[rb], add=True)

            stage(0, 0)
            stage(1, 1)

            def pair_body(t, _):
                k = t * 2
                for r in range(2):
                    process(k + r, r)
                    stage(k + 2 + r, r)
                return 0
            lax.fori_loop(0, NCHUNK // 2, pair_body, 0)
            for r in range(2):
                pltpu.make_async_copy(
                    z_hbm.at[cols[r]], gbufs[r], sems[r]).wait()
        plsc.subcore_barrier()
        pltpu.sync_copy(acc.at[pl.ds(base, RPT)],
                        out_hbm.at[pl.ds(bN + base, RPT)])

        @pl.when(sid == NUM_TECS - 1)
        def _flush_tail():
            pltpu.sync_copy(acc.at[pl.ds(RPT * NUM_TECS, TAIL)],
                            out_hbm.at[pl.ds(bN + RPT * NUM_TECS, TAIL)])
        return 0

    lax.fori_loop(0, B // 2, batch_body, 0)


_sc_spmm = functools.partial(
    pl.kernel,
    out_type=jax.ShapeDtypeStruct((M, U), jnp.float32),
    mesh=plsc.VectorSubcoreMesh(core_axis_name="c", subcore_axis_name="s"),
    scratch_types=[
        pltpu.VMEM_SHARED((N, U), jnp.float32),
        pltpu.VMEM((3, CHUNK), jnp.int32),
        pltpu.VMEM((3, CHUNK), jnp.int32),
        pltpu.VMEM((CHUNK,), jnp.int32),
        pltpu.VMEM((CHUNK,), jnp.int32),
        pltpu.VMEM((CHUNK,), jnp.int32),
        pltpu.VMEM((CHUNK,), jnp.int32),
        pltpu.VMEM((CHUNK, U), jnp.float32),
        pltpu.VMEM((CHUNK, U), jnp.float32),
        pltpu.SemaphoreType.DMA,
        pltpu.SemaphoreType.DMA,
    ],
)(_sc_body)


def _pack_edges(edge_index, values):
    pad = NUM_TECS * NCHUNK * CHUNK - E
    cols = jnp.pad(edge_index[1], (0, pad))
    rows = jnp.pad(edge_index[0], (0, pad))
    vals = jnp.pad(jnp.round(values * VSCALE).astype(jnp.int32), (0, pad))
    packed = jnp.stack([cols, rows, vals], 0)
    packed = packed.reshape(3, NUM_TECS, NCHUNK, CHUNK)
    packed = jnp.pad(packed, ((0, 0), (0, 0), (0, NCF - NCHUNK), (0, 0)))
    return jnp.transpose(packed, (1, 2, 0, 3))


def kernel(x, edge_index0, values0, edge_index1, values1, kernel, bias):
    xf = x.reshape(M, D)
    kw = kernel.reshape(D, 3, U)
    k0 = kw[:, 0, :]
    k1p = kw[:, 1, :]
    k2p = kw[:, 2, :]

    z1, z2 = pl.pallas_call(
        _mm2_body,
        grid=(M // _BM,),
        in_specs=[
            pl.BlockSpec((_BM, D), lambda i: (i, 0)),
            pl.BlockSpec((D, U), lambda i: (0, 0)),
            pl.BlockSpec((D, U), lambda i: (0, 0)),
        ],
        out_specs=[
            pl.BlockSpec((_BM, U), lambda i: (i, 0)),
            pl.BlockSpec((_BM, U), lambda i: (i, 0)),
        ],
        out_shape=[
            jax.ShapeDtypeStruct((M, U), jnp.float32),
            jax.ShapeDtypeStruct((M, U), jnp.float32),
        ],
    )(xf, k1p, k2p)

    s = _sc_spmm(z1, z2,
                 _pack_edges(edge_index0, values0),
                 _pack_edges(edge_index1, values1))

    bias2 = jnp.broadcast_to(bias, (8, U))
    out = pl.pallas_call(
        _mmadd_body,
        grid=(M // _BM,),
        in_specs=[
            pl.BlockSpec((_BM, D), lambda i: (i, 0)),
            pl.BlockSpec((_BM, U), lambda i: (i, 0)),
            pl.BlockSpec((D, U), lambda i: (0, 0)),
            pl.BlockSpec((8, U), lambda i: (0, 0)),
        ],
        out_specs=pl.BlockSpec((_BM, U), lambda i: (i, 0)),
        out_shape=jax.ShapeDtypeStruct((M, U), jnp.float32),
    )(xf, s, k0, bias2)

    return out.reshape(B, N, U)


# speedup vs baseline: 1.1526x; 1.1526x over previous
import functools

import jax
import jax.numpy as jnp
import numpy as np
from jax import lax
from jax.experimental import pallas as pl
from jax.experimental.pallas import tpu as pltpu
from jax.experimental.pallas import tpu_sc as plsc

B = 8
N = 10000
D = 128
U = 128
E = 320000
M = B * N

NUM_TECS = 16
CHUNK = 128
NCHUNK = 158
NCF = NCHUNK + 1
EPAD = NUM_TECS * NCF * CHUNK
RPT = 624
TAIL = N - RPT * NUM_TECS
VSCALE = float(2 ** 24)

_BM = 2000

_PERM = np.empty(U, np.int32)
for _j in range(U // 32):
    for _t in range(16):
        _PERM[32 * _j + 2 * _t] = 32 * _j + _t
        _PERM[32 * _j + 2 * _t + 1] = 32 * _j + 16 + _t


def _mm2_body(x_ref, k1_ref, k2_ref, z1_ref, z2_ref):
    xb = x_ref[...]
    z1_ref[...] = jnp.dot(xb, k1_ref[...], preferred_element_type=jnp.float32)
    z2_ref[...] = jnp.dot(xb, k2_ref[...], preferred_element_type=jnp.float32)


def _mmadd_body(x_ref, s_ref, k0_ref, b_ref, o_ref):
    o_ref[...] = (jnp.dot(x_ref[...], k0_ref[...],
                          preferred_element_type=jnp.float32)
                  + s_ref[...] + b_ref[...][0:1, :])


def _sc_body(z1_hbm, z2_hbm, p0_hbm, p1_hbm, out_hbm,
             acc, ring0, ring1, col0, col1,
             row0, row1, gbuf0, gbuf1, sem0, sem1):
    cid = lax.axis_index("c")
    sid = lax.axis_index("s")
    base = sid * RPT
    rings = (ring0, ring1)
    cols = (col0, col1)
    rows = (row0, row1)
    gbufs = (gbuf0, gbuf1)
    sems = (sem0, sem1)

    def batch_body(bi, _):
        b = cid * (B // 2) + bi
        bN = b * N

        def zloop(i, _):
            for j in range(U // 16):
                gbuf0[i, pl.ds(j * 16, 16)] = jnp.zeros((16,), jnp.float32)
            return 0
        lax.fori_loop(0, CHUNK, zloop, 0)
        for k in range(RPT // CHUNK):
            pltpu.sync_copy(gbuf0, acc.at[pl.ds(base + k * CHUNK, CHUNK)])
        rem = RPT % CHUNK
        if rem:
            pltpu.sync_copy(gbuf0.at[pl.ds(0, rem)],
                            acc.at[pl.ds(base + RPT - rem, rem)])

        @pl.when(sid == NUM_TECS - 1)
        def _zero_tail():
            pltpu.sync_copy(gbuf0.at[pl.ds(0, TAIL)],
                            acc.at[pl.ds(RPT * NUM_TECS, TAIL)])
        plsc.subcore_barrier()

        for z_hbm, p_hbm in ((z1_hbm, p0_hbm), (z2_hbm, p1_hbm)):

            def stage(k, r, z_hbm=z_hbm, p_hbm=p_hbm, bN=bN):
                rg = rings[r]
                cb = cols[r]
                pltpu.sync_copy(p_hbm.at[sid, k], rg)
                for j in range(CHUNK // 16):
                    cb[pl.ds(j * 16, 16)] = rg[0, pl.ds(j * 16, 16)] + bN
                pltpu.async_copy(z_hbm.at[cb], gbufs[r], sems[r])

            def process(k, r, z_hbm=z_hbm):
                pltpu.make_async_copy(
                    z_hbm.at[cols[r]], gbufs[r], sems[r]).wait()
                rg = rings[r]
                gb = gbufs[r]

                def srow(t, _):
                    valv = (rg[2, pl.ds(t * 16, 16)].astype(jnp.float32)
                            * (1.0 / VSCALE))
                    for i in range(16):
                        row = t * 16 + i
                        v = valv[i]
                        for j in range(U // 16):
                            gb[row, pl.ds(j * 16, 16)] = (
                                gb[row, pl.ds(j * 16, 16)] * v)
                    return 0
                lax.fori_loop(0, CHUNK // 16, srow, 0)
                rb = rows[r]
                for j in range(CHUNK // 16):
                    rb[pl.ds(j * 16, 16)] = rg[1, pl.ds(j * 16, 16)]
                pltpu.sync_copy(gb, acc.at[rb], add=True)

            stage(0, 0)

            def pair_body(t, _):
                k = t * 2
                stage(k + 1, 1)
                process(k, 0)
                stage(k + 2, 0)
                process(k + 1, 1)
                return 0
            lax.fori_loop(0, NCHUNK // 2, pair_body, 0)
            pltpu.make_async_copy(
                z_hbm.at[cols[0]], gbufs[0], sems[0]).wait()
        plsc.subcore_barrier()
        pltpu.sync_copy(acc.at[pl.ds(base, RPT)],
                        out_hbm.at[pl.ds(bN + base, RPT)])

        @pl.when(sid == NUM_TECS - 1)
        def _flush_tail():
            pltpu.sync_copy(acc.at[pl.ds(RPT * NUM_TECS, TAIL)],
                            out_hbm.at[pl.ds(bN + RPT * NUM_TECS, TAIL)])
        return 0

    lax.fori_loop(0, B // 2, batch_body, 0)


_sc_spmm = functools.partial(
    pl.kernel,
    out_type=jax.ShapeDtypeStruct((M, U), jnp.float32),
    mesh=plsc.VectorSubcoreMesh(core_axis_name="c", subcore_axis_name="s"),
    scratch_types=[
        pltpu.VMEM_SHARED((N, U), jnp.float32),
        pltpu.VMEM((3, CHUNK), jnp.int32),
        pltpu.VMEM((3, CHUNK), jnp.int32),
        pltpu.VMEM((CHUNK,), jnp.int32),
        pltpu.VMEM((CHUNK,), jnp.int32),
        pltpu.VMEM((CHUNK,), jnp.int32),
        pltpu.VMEM((CHUNK,), jnp.int32),
        pltpu.VMEM((CHUNK, U), jnp.float32),
        pltpu.VMEM((CHUNK, U), jnp.float32),
        pltpu.SemaphoreType.DMA,
        pltpu.SemaphoreType.DMA,
    ],
)(_sc_body)


def _pack_edges(edge_index, values):
    pad = NUM_TECS * NCHUNK * CHUNK - E
    cols = jnp.pad(edge_index[1], (0, pad))
    rows = jnp.pad(edge_index[0], (0, pad))
    vals = jnp.pad(jnp.round(values * VSCALE).astype(jnp.int32), (0, pad))
    packed = jnp.stack([cols, rows, vals], 0)
    packed = packed.reshape(3, NUM_TECS, NCHUNK, CHUNK)
    packed = jnp.pad(packed, ((0, 0), (0, 0), (0, NCF - NCHUNK), (0, 0)))
    return jnp.transpose(packed, (1, 2, 0, 3))


def kernel(x, edge_index0, values0, edge_index1, values1, kernel, bias):
    xf = x.reshape(M, D)
    kw = kernel.reshape(D, 3, U)
    k0 = kw[:, 0, :]
    k1p = kw[:, 1, :]
    k2p = kw[:, 2, :]

    z1, z2 = pl.pallas_call(
        _mm2_body,
        grid=(M // _BM,),
        in_specs=[
            pl.BlockSpec((_BM, D), lambda i: (i, 0)),
            pl.BlockSpec((D, U), lambda i: (0, 0)),
            pl.BlockSpec((D, U), lambda i: (0, 0)),
        ],
        out_specs=[
            pl.BlockSpec((_BM, U), lambda i: (i, 0)),
            pl.BlockSpec((_BM, U), lambda i: (i, 0)),
        ],
        out_shape=[
            jax.ShapeDtypeStruct((M, U), jnp.float32),
            jax.ShapeDtypeStruct((M, U), jnp.float32),
        ],
    )(xf, k1p, k2p)

    s = _sc_spmm(z1, z2,
                 _pack_edges(edge_index0, values0),
                 _pack_edges(edge_index1, values1))

    bias2 = jnp.broadcast_to(bias, (8, U))
    out = pl.pallas_call(
        _mmadd_body,
        grid=(M // _BM,),
        in_specs=[
            pl.BlockSpec((_BM, D), lambda i: (i, 0)),
            pl.BlockSpec((_BM, U), lambda i: (i, 0)),
            pl.BlockSpec((D, U), lambda i: (0, 0)),
            pl.BlockSpec((8, U), lambda i: (0, 0)),
        ],
        out_specs=pl.BlockSpec((_BM, U), lambda i: (i, 0)),
        out_shape=jax.ShapeDtypeStruct((M, U), jnp.float32),
    )(xf, s, k0, bias2)

    return out.reshape(B, N, U)
